# baseline (speedup 1.0000x reference)
import functools

import jax
import jax.numpy as jnp
from jax import lax
from jax.experimental import pallas as pl
from jax.experimental.pallas import tpu as pltpu
from jax.experimental.pallas import tpu_sc as plsc

N_TOK = 4096
D = 1024
FF_S = 4096
FF_B = 16384
TB = 512
F = 2048
NT = N_TOK // TB
NF_S = FF_S // F
NF_B = FF_B // F
NF = NF_S + NF_B

MR = 32
MC = 128

NC, NS = 2, 16
NW = NC * NS
CHUNK = N_TOK // NW
SUB = 32
NSUB = CHUNK // SUB


def _gelu(h):
    return 0.5 * h * (1.0 + jax.lax.erf(h * 0.7071067811865476))


def _route_body(mf_ref, dst_ref, ns_ref):
    mf = mf_ref[...]
    mbf = mf.astype(jnp.bfloat16)
    jj = lax.broadcasted_iota(jnp.int32, (MC, MC), 0)
    kk = lax.broadcasted_iota(jnp.int32, (MC, MC), 1)
    tri = (jj <= kk).astype(jnp.bfloat16)
    c_row = jnp.dot(mbf, tri, preferred_element_type=jnp.float32)
    tot = c_row[:, MC - 1:MC]
    rr = lax.broadcasted_iota(jnp.int32, (MR, MR), 0)
    cc = lax.broadcasted_iota(jnp.int32, (MR, MR), 1)
    low = (rr > cc).astype(jnp.bfloat16)
    rowpre = jnp.dot(low, tot.astype(jnp.bfloat16),
                     preferred_element_type=jnp.float32)
    ones_incl = c_row + rowpre
    ones_before = ones_incl - mf
    gidx = (lax.broadcasted_iota(jnp.int32, (MR, MC), 0) * MC
            + lax.broadcasted_iota(jnp.int32, (MR, MC), 1)).astype(jnp.float32)
    zeros_before = gidx - ones_before
    n_ones = jnp.sum(mf)
    ns = jnp.float32(N_TOK) - n_ones
    dstf = jnp.where(mf > 0.5, ns + ones_before, zeros_before)
    dst_ref[...] = dstf.astype(jnp.int32)
    ns_ref[0] = ns.astype(jnp.int32)


def _route(maskf):
    return pl.pallas_call(
        _route_body,
        in_specs=[pl.BlockSpec((MR, MC), lambda: (0, 0))],
        out_specs=[
            pl.BlockSpec((MR, MC), lambda: (0, 0)),
            pl.BlockSpec(memory_space=pltpu.SMEM),
        ],
        out_shape=[
            jax.ShapeDtypeStruct((MR, MC), jnp.int32),
            jax.ShapeDtypeStruct((1,), jnp.int32),
        ],
    )(maskf)


@functools.lru_cache(maxsize=None)
def _sc_kernels():
    mesh = plsc.VectorSubcoreMesh(
        core_axis_name="c", subcore_axis_name="s",
        num_cores=NC, num_subcores=NS)

    @functools.partial(
        pl.kernel, mesh=mesh,
        out_type=jax.ShapeDtypeStruct((N_TOK, D), jnp.float32),
        scratch_types=[
            pltpu.VMEM((NSUB, SUB), jnp.int32),
            pltpu.VMEM((SUB, D), jnp.float32),
            pltpu.SemaphoreType.DMA,
        ],
    )
    def _sc_scatter(x_hbm, idx3_hbm, xs_hbm, idx_v, buf_v, sem):
        wid = lax.axis_index("s") * NC + lax.axis_index("c")
        base = wid * CHUNK
        pltpu.sync_copy(idx3_hbm.at[wid], idx_v)
        for j in range(NSUB):
            pltpu.sync_copy(x_hbm.at[pl.ds(base + j * SUB, SUB)], buf_v)
            pltpu.async_copy(buf_v, xs_hbm.at[idx_v.at[j]], sem).wait()

    @functools.partial(
        pl.kernel, mesh=mesh,
        out_type=jax.ShapeDtypeStruct((N_TOK, D), jnp.float32),
        scratch_types=[
            pltpu.VMEM((NSUB, SUB), jnp.int32),
            pltpu.VMEM((SUB, D), jnp.float32),
            pltpu.SemaphoreType.DMA,
        ],
    )
    def _sc_gather(os_hbm, idx3_hbm, out_hbm, idx_v, buf_v, sem):
        wid = lax.axis_index("s") * NC + lax.axis_index("c")
        base = wid * CHUNK
        pltpu.sync_copy(idx3_hbm.at[wid], idx_v)
        for j in range(NSUB):
            pltpu.async_copy(os_hbm.at[idx_v.at[j]], buf_v, sem).wait()
            pltpu.sync_copy(buf_v, out_hbm.at[pl.ds(base + j * SUB, SUB)])

    return _sc_scatter, _sc_gather


FC = 512
NCH = F // FC


def _expert_tile(x, wf_ref, wp_ref, bias_row):
    p = None
    for c in range(NCH):
        h = jnp.dot(x, wf_ref[:, c * FC:(c + 1) * FC],
                    preferred_element_type=jnp.float32)
        h = _gelu(h + bias_row[:, c * FC:(c + 1) * FC]).astype(jnp.bfloat16)
        pc = jnp.dot(h, wp_ref[c * FC:(c + 1) * FC, :],
                     preferred_element_type=jnp.float32)
        p = pc if p is None else p + pc
    return p


def _moe_body(ns_ref, x_ref, wfs_ref, wfb_ref, wps_ref, wpb_ref, bfc_ref,
              bps_ref, bpb_ref, out_ref, acc_ref):
    f = pl.program_id(0)
    t = pl.program_id(1)
    ns = ns_ref[0]
    row = t * TB + lax.broadcasted_iota(jnp.int32, (TB, 1), 0)
    coeff = (row >= ns).astype(jnp.float32)
    boundary = (t * TB < ns) & ((t + 1) * TB > ns)

    @pl.when(f == 0)
    def _init():
        acc_ref[t] = (1.0 - coeff) * bps_ref[...] + coeff * bpb_ref[...]

    @pl.when((f < NF_S) & (t * TB < ns))
    def _small():
        p = _expert_tile(x_ref[...], wfs_ref, wps_ref, bfc_ref[0])

        @pl.when(boundary)
        def _():
            acc_ref[t] += (1.0 - coeff) * p

        @pl.when(jnp.logical_not(boundary))
        def _():
            acc_ref[t] += p

    @pl.when((f >= NF_S) & ((t + 1) * TB > ns))
    def _big():
        p = _expert_tile(x_ref[...], wfb_ref, wpb_ref, bfc_ref[0])

        @pl.when(boundary)
        def _():
            acc_ref[t] += coeff * p

        @pl.when(jnp.logical_not(boundary))
        def _():
            acc_ref[t] += p

    @pl.when(f == NF - 1)
    def _emit():
        out_ref[...] = acc_ref[t]


def _moe_mlp(ns, xb, wfs, wfb, wps, wpb, bfc, bps, bpb):
    def _x_idx(f, t, n):
        ns = n[0]
        needed = jnp.where(f < NF_S, t * TB < ns, (t + 1) * TB > ns)
        return (jnp.where(needed, t, 0), 0)

    def _out_idx(f, t, n):
        return (jnp.where(f == NF - 1, t, 0), 0)

    grid_spec = pltpu.PrefetchScalarGridSpec(
        num_scalar_prefetch=1,
        grid=(NF, NT),
        in_specs=[
            pl.BlockSpec((TB, D), _x_idx),
            pl.BlockSpec((D, F), lambda f, t, n: (0, jnp.minimum(f, NF_S - 1))),
            pl.BlockSpec((D, F), lambda f, t, n: (0, jnp.clip(f - NF_S, 0, NF_B - 1))),
            pl.BlockSpec((F, D), lambda f, t, n: (jnp.minimum(f, NF_S - 1), 0)),
            pl.BlockSpec((F, D), lambda f, t, n: (jnp.clip(f - NF_S, 0, NF_B - 1), 0)),
            pl.BlockSpec((1, 1, F), lambda f, t, n: (f, 0, 0)),
            pl.BlockSpec((1, D), lambda f, t, n: (0, 0)),
            pl.BlockSpec((1, D), lambda f, t, n: (0, 0)),
        ],
        out_specs=pl.BlockSpec((TB, D), _out_idx),
        scratch_shapes=[pltpu.VMEM((NT, TB, D), jnp.float32)],
    )
    return pl.pallas_call(
        _moe_body,
        grid_spec=grid_spec,
        out_shape=jax.ShapeDtypeStruct((N_TOK, D), jnp.float32),
        compiler_params=pltpu.CompilerParams(
            dimension_semantics=("arbitrary", "arbitrary")),
    )(ns, xb, wfs, wfb, wps, wpb, bfc, bps, bpb)


def kernel(x, W_fc_s, b_fc_s, W_proj_s, b_proj_s, W_fc_b, b_fc_b,
           W_proj_b, b_proj_b, mask):
    maskf = mask.astype(jnp.float32).reshape(MR, MC)
    dst, ns = _route(maskf)
    idx3 = dst.reshape(NW, NSUB, SUB)

    sc_scatter, sc_gather = _sc_kernels()
    xs = sc_scatter(x, idx3)
    xsb = xs.astype(jnp.bfloat16)

    wfs = W_fc_s.astype(jnp.bfloat16)
    wfb = W_fc_b.astype(jnp.bfloat16)
    wps = W_proj_s.astype(jnp.bfloat16)
    wpb = W_proj_b.astype(jnp.bfloat16)
    bfc = jnp.concatenate([b_fc_s, b_fc_b]).reshape(NF, 1, F)
    bps = b_proj_s.reshape(1, D)
    bpb = b_proj_b.reshape(1, D)

    out_sorted = _moe_mlp(ns, xsb, wfs, wfb, wps, wpb, bfc, bps, bpb)
    return sc_gather(out_sorted, idx3)


# speedup vs baseline: 1.0372x; 1.0372x over previous
import functools

import jax
import jax.numpy as jnp
from jax import lax
from jax.experimental import pallas as pl
from jax.experimental.pallas import tpu as pltpu
from jax.experimental.pallas import tpu_sc as plsc

N_TOK = 4096
D = 1024
FF_S = 4096
FF_B = 16384
TB = 512
F = 2048
NT = N_TOK // TB
NF_S = FF_S // F
NF_B = FF_B // F
NF = NF_S + NF_B

MR = 32
MC = 128

NC, NS = 2, 16
NW = NC * NS
CHUNK = N_TOK // NW
SUB = 64
NSUB = CHUNK // SUB


def _gelu(h):
    return 0.5 * h * (1.0 + jax.lax.erf(h * 0.7071067811865476))


def _route_body(mf_ref, dst_ref, ns_ref):
    mf = mf_ref[...]
    mbf = mf.astype(jnp.bfloat16)
    jj = lax.broadcasted_iota(jnp.int32, (MC, MC), 0)
    kk = lax.broadcasted_iota(jnp.int32, (MC, MC), 1)
    tri = (jj <= kk).astype(jnp.bfloat16)
    c_row = jnp.dot(mbf, tri, preferred_element_type=jnp.float32)
    tot = c_row[:, MC - 1:MC]
    rr = lax.broadcasted_iota(jnp.int32, (MR, MR), 0)
    cc = lax.broadcasted_iota(jnp.int32, (MR, MR), 1)
    low = (rr > cc).astype(jnp.bfloat16)
    rowpre = jnp.dot(low, tot.astype(jnp.bfloat16),
                     preferred_element_type=jnp.float32)
    ones_incl = c_row + rowpre
    ones_before = ones_incl - mf
    gidx = (lax.broadcasted_iota(jnp.int32, (MR, MC), 0) * MC
            + lax.broadcasted_iota(jnp.int32, (MR, MC), 1)).astype(jnp.float32)
    zeros_before = gidx - ones_before
    n_ones = jnp.sum(mf)
    ns = jnp.float32(N_TOK) - n_ones
    dstf = jnp.where(mf > 0.5, ns + ones_before, zeros_before)
    dst_ref[...] = dstf.astype(jnp.int32)
    ns_ref[0] = ns.astype(jnp.int32)


def _route(maskf):
    return pl.pallas_call(
        _route_body,
        in_specs=[pl.BlockSpec((MR, MC), lambda: (0, 0))],
        out_specs=[
            pl.BlockSpec((MR, MC), lambda: (0, 0)),
            pl.BlockSpec(memory_space=pltpu.SMEM),
        ],
        out_shape=[
            jax.ShapeDtypeStruct((MR, MC), jnp.int32),
            jax.ShapeDtypeStruct((1,), jnp.int32),
        ],
    )(maskf)


@functools.lru_cache(maxsize=None)
def _sc_kernels():
    mesh = plsc.VectorSubcoreMesh(
        core_axis_name="c", subcore_axis_name="s",
        num_cores=NC, num_subcores=NS)

    @functools.partial(
        pl.kernel, mesh=mesh,
        out_type=jax.ShapeDtypeStruct((N_TOK, D), jnp.float32),
        scratch_types=[
            pltpu.VMEM((NSUB, SUB), jnp.int32),
            pltpu.VMEM((SUB, D), jnp.float32),
            pltpu.SemaphoreType.DMA,
        ],
    )
    def _sc_scatter(x_hbm, idx3_hbm, xs_hbm, idx_v, buf_v, sem):
        wid = lax.axis_index("s") * NC + lax.axis_index("c")
        base = wid * CHUNK
        pltpu.sync_copy(idx3_hbm.at[wid], idx_v)
        for j in range(NSUB):
            pltpu.sync_copy(x_hbm.at[pl.ds(base + j * SUB, SUB)], buf_v)
            pltpu.async_copy(buf_v, xs_hbm.at[idx_v.at[j]], sem).wait()

    @functools.partial(
        pl.kernel, mesh=mesh,
        out_type=jax.ShapeDtypeStruct((N_TOK, D), jnp.float32),
        scratch_types=[
            pltpu.VMEM((NSUB, SUB), jnp.int32),
            pltpu.VMEM((SUB, D), jnp.float32),
            pltpu.SemaphoreType.DMA,
        ],
    )
    def _sc_gather(os_hbm, idx3_hbm, out_hbm, idx_v, buf_v, sem):
        wid = lax.axis_index("s") * NC + lax.axis_index("c")
        base = wid * CHUNK
        pltpu.sync_copy(idx3_hbm.at[wid], idx_v)
        for j in range(NSUB):
            pltpu.async_copy(os_hbm.at[idx_v.at[j]], buf_v, sem).wait()
            pltpu.sync_copy(buf_v, out_hbm.at[pl.ds(base + j * SUB, SUB)])

    return _sc_scatter, _sc_gather


FC = 1024
NCH = F // FC


def _expert_tile(x, wf_ref, wp_ref, bias_row):
    p = None
    for c in range(NCH):
        h = jnp.dot(x, wf_ref[:, c * FC:(c + 1) * FC],
                    preferred_element_type=jnp.float32)
        h = _gelu(h + bias_row[:, c * FC:(c + 1) * FC]).astype(jnp.bfloat16)
        pc = jnp.dot(h, wp_ref[c * FC:(c + 1) * FC, :],
                     preferred_element_type=jnp.float32)
        p = pc if p is None else p + pc
    return p


def _moe_body(ns_ref, x_ref, wfs_ref, wfb_ref, wps_ref, wpb_ref, bfc_ref,
              bps_ref, bpb_ref, out_ref, acc_ref):
    f = pl.program_id(0)
    t = pl.program_id(1)
    ns = ns_ref[0]
    row = t * TB + lax.broadcasted_iota(jnp.int32, (TB, 1), 0)
    coeff = (row >= ns).astype(jnp.float32)
    boundary = (t * TB < ns) & ((t + 1) * TB > ns)

    @pl.when(f == 0)
    def _init():
        acc_ref[t] = (1.0 - coeff) * bps_ref[...] + coeff * bpb_ref[...]

    @pl.when((f < NF_S) & (t * TB < ns))
    def _small():
        p = _expert_tile(x_ref[...], wfs_ref, wps_ref, bfc_ref[0])

        @pl.when(boundary)
        def _():
            acc_ref[t] += (1.0 - coeff) * p

        @pl.when(jnp.logical_not(boundary))
        def _():
            acc_ref[t] += p

    @pl.when((f >= NF_S) & ((t + 1) * TB > ns))
    def _big():
        p = _expert_tile(x_ref[...], wfb_ref, wpb_ref, bfc_ref[0])

        @pl.when(boundary)
        def _():
            acc_ref[t] += coeff * p

        @pl.when(jnp.logical_not(boundary))
        def _():
            acc_ref[t] += p

    @pl.when(f == NF - 1)
    def _emit():
        out_ref[...] = acc_ref[t]


def _moe_mlp(ns, xb, wfs, wfb, wps, wpb, bfc, bps, bpb):
    def _x_idx(f, t, n):
        ns = n[0]
        needed = jnp.where(f < NF_S, t * TB < ns, (t + 1) * TB > ns)
        return (jnp.where(needed, t, 0), 0)

    def _out_idx(f, t, n):
        return (jnp.where(f == NF - 1, t, 0), 0)

    grid_spec = pltpu.PrefetchScalarGridSpec(
        num_scalar_prefetch=1,
        grid=(NF, NT),
        in_specs=[
            pl.BlockSpec((TB, D), _x_idx),
            pl.BlockSpec((D, F), lambda f, t, n: (0, jnp.minimum(f, NF_S - 1))),
            pl.BlockSpec((D, F), lambda f, t, n: (0, jnp.clip(f - NF_S, 0, NF_B - 1))),
            pl.BlockSpec((F, D), lambda f, t, n: (jnp.minimum(f, NF_S - 1), 0)),
            pl.BlockSpec((F, D), lambda f, t, n: (jnp.clip(f - NF_S, 0, NF_B - 1), 0)),
            pl.BlockSpec((1, 1, F), lambda f, t, n: (f, 0, 0)),
            pl.BlockSpec((1, D), lambda f, t, n: (0, 0)),
            pl.BlockSpec((1, D), lambda f, t, n: (0, 0)),
        ],
        out_specs=pl.BlockSpec((TB, D), _out_idx),
        scratch_shapes=[pltpu.VMEM((NT, TB, D), jnp.float32)],
    )
    return pl.pallas_call(
        _moe_body,
        grid_spec=grid_spec,
        out_shape=jax.ShapeDtypeStruct((N_TOK, D), jnp.float32),
        compiler_params=pltpu.CompilerParams(
            dimension_semantics=("arbitrary", "arbitrary")),
    )(ns, xb, wfs, wfb, wps, wpb, bfc, bps, bpb)


def kernel(x, W_fc_s, b_fc_s, W_proj_s, b_proj_s, W_fc_b, b_fc_b,
           W_proj_b, b_proj_b, mask):
    maskf = mask.astype(jnp.float32).reshape(MR, MC)
    dst, ns = _route(maskf)
    idx3 = dst.reshape(NW, NSUB, SUB)

    sc_scatter, sc_gather = _sc_kernels()
    xsb = sc_scatter(x, idx3).astype(jnp.bfloat16)

    wfs = W_fc_s.astype(jnp.bfloat16)
    wfb = W_fc_b.astype(jnp.bfloat16)
    wps = W_proj_s.astype(jnp.bfloat16)
    wpb = W_proj_b.astype(jnp.bfloat16)
    bfc = jnp.concatenate([b_fc_s, b_fc_b]).reshape(NF, 1, F)
    bps = b_proj_s.reshape(1, D)
    bpb = b_proj_b.reshape(1, D)

    out_sorted = _moe_mlp(ns, xsb, wfs, wfb, wps, wpb, bfc, bps, bpb)
    return sc_gather(out_sorted, idx3)
